# Initial kernel scaffold; baseline (speedup 1.0000x reference)
import jax
import jax.numpy as jnp
from jax.experimental import pallas as pl


def kernel(poss_node, poss_edge, weights, groundTruth, mask, edges):
    raise NotImplementedError("write your pallas kernel here")


# speedup vs baseline: 5.9818x; 5.9818x over previous
import functools

import jax
import jax.numpy as jnp
from jax import lax
from jax.experimental import pallas as pl
from jax.experimental.pallas import tpu as pltpu
from jax.experimental.pallas import tpu_sc as plsc

SEMI_LAMBDA = 0.5
N_NODES = 100000
N_EDGES = 6400000
N_CLASSES = 16

NC = 2
NS = 16
L = 16
E_PER_TILE = N_EDGES // NS
CHUNK = 4000
CHUNK_W = 2 * CHUNK
NCHUNK = E_PER_TILE // CHUNK
STEPS = CHUNK // L

_sc_mesh = plsc.VectorSubcoreMesh(core_axis_name="c", subcore_axis_name="s")


@functools.partial(
    pl.kernel,
    mesh=_sc_mesh,
    compiler_params=pltpu.CompilerParams(needs_layout_passes=False),
    out_type=jax.ShapeDtypeStruct((NC, NS, L), jnp.float32),
    scratch_types=[
        pltpu.VMEM((N_NODES,), jnp.float32),
        pltpu.VMEM((CHUNK_W,), jnp.int32),
        pltpu.VMEM((CHUNK_W,), jnp.int32),
        pltpu.VMEM((L,), jnp.float32),
        pltpu.SemaphoreType.DMA,
        pltpu.SemaphoreType.DMA,
    ],
)
def _sc_edge(pe_t_hbm, edges_hbm, out_hbm, table_v, ebuf0, ebuf1, accv, sem_t, sem_e):
    c = lax.axis_index("c")
    s = lax.axis_index("s")
    pltpu.async_copy(pe_t_hbm.at[c], table_v, sem_t).wait()

    base = s * (2 * E_PER_TILE)
    two_iota = lax.iota(jnp.int32, L) * 2

    bufs = (ebuf0, ebuf1)

    def start(g, buf):
        pltpu.async_copy(
            edges_hbm.at[pl.ds(base + g * CHUNK_W, CHUNK_W)], bufs[buf], sem_e
        )

    def wait(g, buf):
        pltpu.make_async_copy(
            edges_hbm.at[pl.ds(base + g * CHUNK_W, CHUNK_W)], bufs[buf], sem_e
        ).wait()

    start(0, 0)

    def make_step(eb):
        def body(j, acc):
            si = two_iota + j * (2 * L)
            srcs = plsc.load_gather(eb, [si])
            dsts = plsc.load_gather(eb, [si + 1])
            av = plsc.load_gather(table_v, [srcs])
            bv = plsc.load_gather(table_v, [dsts])
            d = av - bv
            return acc + d * d

        return body

    def outer(g2, acc):
        for b in range(2):
            g = 2 * g2 + b

            @pl.when(g + 1 < NCHUNK)
            def _():
                start(g + 1, 1 - b)

            wait(g, b)
            acc = lax.fori_loop(0, STEPS, make_step(bufs[b]), acc)
        return acc

    acc = lax.fori_loop(
        0, NCHUNK // 2, outer, jnp.zeros((L,), jnp.float32)
    )
    accv[...] = acc
    pltpu.sync_copy(accv, out_hbm.at[c, s])


BLK = 2000
GRID = N_NODES // BLK


def _tc_body(pn_ref, gt_ref, m_ref, pe_ref, out_ref, acc_ref):
    i = pl.program_id(0)

    @pl.when(i == 0)
    def _():
        acc_ref[0] = 0.0
        acc_ref[1] = 0.0

    p = pn_ref[...]
    gt = gt_ref[0, 0, :]
    m = m_ref[0, 0, :].astype(jnp.float32)
    cols = lax.broadcasted_iota(jnp.int32, (BLK, N_CLASSES), 1)
    pv = jnp.sum(jnp.where(cols == gt[:, None], p, 0.0), axis=1)
    acc_ref[0] += jnp.sum(jnp.log(pv) * m)
    acc_ref[1] += jnp.sum(m)

    @pl.when(i == GRID - 1)
    def _():
        t = jnp.float32(N_NODES) - jnp.sum(pe_ref[1, :])
        out_ref[0] = -acc_ref[0] / acc_ref[1]
        out_ref[1] = t


_tc_nll = pl.pallas_call(
    _tc_body,
    grid=(GRID,),
    in_specs=[
        pl.BlockSpec((BLK, N_CLASSES), lambda i: (i, 0)),
        pl.BlockSpec((1, 1, BLK), lambda i: (i, 0, 0)),
        pl.BlockSpec((1, 1, BLK), lambda i: (i, 0, 0)),
        pl.BlockSpec((NC, N_NODES), lambda i: (0, 0)),
    ],
    out_specs=pl.BlockSpec(memory_space=pltpu.SMEM),
    out_shape=jax.ShapeDtypeStruct((2,), jnp.float32),
    scratch_shapes=[pltpu.SMEM((2,), jnp.float32)],
)


def kernel(poss_node, poss_edge, weights, groundTruth, mask, edges):
    pe_t = poss_edge.T
    edges_flat = edges.reshape(-1)
    gt3 = groundTruth.reshape(GRID, 1, BLK).astype(jnp.int32)
    m3 = mask.reshape(GRID, 1, BLK).astype(jnp.int32)
    parts = _sc_edge(pe_t, edges_flat)
    tc = _tc_nll(poss_node, gt3, m3, pe_t)
    s_total = jnp.sum(parts)
    return tc[0] + jnp.float32(SEMI_LAMBDA) * s_total * tc[1]


# speedup vs baseline: 404.2522x; 67.5804x over previous
import functools

import jax
import jax.numpy as jnp
from jax import lax
from jax.experimental import pallas as pl
from jax.experimental.pallas import tpu as pltpu
from jax.experimental.pallas import tpu_sc as plsc

SEMI_LAMBDA = 0.5
N_NODES = 100000
N_EDGES = 6400000
N_CLASSES = 16

NC = 2
NS = 16
L = 16
E_PER_TILE = N_EDGES // NS
CHUNK = 3200
CHUNK_W = 2 * CHUNK
NCHUNK = E_PER_TILE // CHUNK
BLOCKS = CHUNK // 128
VPB = 128 // L

_sc_mesh = plsc.VectorSubcoreMesh(core_axis_name="c", subcore_axis_name="s")


@functools.partial(
    pl.kernel,
    mesh=_sc_mesh,
    compiler_params=pltpu.CompilerParams(needs_layout_passes=False),
    out_type=jax.ShapeDtypeStruct((NC, NS, L), jnp.float32),
    scratch_types=[
        pltpu.VMEM((N_NODES,), jnp.float32),
        pltpu.VMEM((CHUNK_W,), jnp.int32),
        pltpu.VMEM((CHUNK_W,), jnp.int32),
        pltpu.VMEM((L,), jnp.float32),
        pltpu.SemaphoreType.DMA,
        pltpu.SemaphoreType.DMA,
    ],
)
def _sc_edge(pe_t_hbm, edges_hbm, out_hbm, table_v, ebuf0, ebuf1, accv, sem_t, sem_e):
    c = lax.axis_index("c")
    s = lax.axis_index("s")
    pltpu.async_copy(pe_t_hbm.at[c], table_v, sem_t).wait()

    base = s * (2 * E_PER_TILE)
    bufs = (ebuf0, ebuf1)

    def start(g, buf):
        pltpu.async_copy(
            edges_hbm.at[pl.ds(base + g * CHUNK_W, CHUNK_W)], bufs[buf], sem_e
        )

    def wait(g, buf):
        pltpu.make_async_copy(
            edges_hbm.at[pl.ds(base + g * CHUNK_W, CHUNK_W)], bufs[buf], sem_e
        ).wait()

    start(0, 0)

    def make_step(eb):
        def body(blk, acc):
            off = blk * 256
            for t in range(VPB):
                srcs = eb[pl.ds(off + t * L, L)]
                dsts = eb[pl.ds(off + 128 + t * L, L)]
                av = plsc.load_gather(table_v, [srcs])
                bv = plsc.load_gather(table_v, [dsts])
                d = av - bv
                acc = acc + d * d
            return acc

        return body

    def do_chunk(g, b, acc):
        @pl.when(g + 1 < NCHUNK)
        def _():
            start(g + 1, 1 - b)

        wait(g, b)
        return lax.fori_loop(0, BLOCKS, make_step(bufs[b]), acc)

    def outer(g2, acc):
        acc = do_chunk(2 * g2, 0, acc)
        acc = do_chunk(2 * g2 + 1, 1, acc)
        return acc

    acc = lax.fori_loop(
        0, NCHUNK // 2, outer, jnp.zeros((L,), jnp.float32)
    )
    acc = do_chunk(NCHUNK - 1, 0, acc)
    accv[...] = acc
    pltpu.sync_copy(accv, out_hbm.at[c, s])


def _tc_body(pn_ref, gt_ref, m_ref, pe_ref, out_ref):
    p = pn_ref[...]
    gt = gt_ref[...]
    m = m_ref[...].astype(jnp.float32)
    rows = lax.broadcasted_iota(jnp.int32, (N_CLASSES, N_NODES), 0)
    pv = jnp.sum(jnp.where(rows == gt[None, :], p, 0.0), axis=0)
    nll = -jnp.sum(jnp.log(pv) * m) / jnp.sum(m)
    t = jnp.float32(N_NODES) - jnp.sum(pe_ref[1, :])
    out_ref[0] = nll
    out_ref[1] = t


_tc_nll = pl.pallas_call(
    _tc_body,
    out_specs=pl.BlockSpec(memory_space=pltpu.SMEM),
    out_shape=jax.ShapeDtypeStruct((2,), jnp.float32),
)


def kernel(poss_node, poss_edge, weights, groundTruth, mask, edges):
    pe_t = poss_edge.T
    pn_t = poss_node.T
    gt = groundTruth.astype(jnp.int32)
    m = mask.astype(jnp.int32)
    edges_n = edges.reshape(N_EDGES // 128, 128, 2).transpose(0, 2, 1).reshape(-1)
    parts = _sc_edge(pe_t, edges_n)
    tc = _tc_nll(pn_t, gt, m, pe_t)
    s_total = jnp.sum(parts)
    return tc[0] + jnp.float32(SEMI_LAMBDA) * s_total * tc[1]


# speedup vs baseline: 596.3290x; 1.4751x over previous
import functools

import jax
import jax.numpy as jnp
from jax import lax
from jax.experimental import pallas as pl
from jax.experimental.pallas import tpu as pltpu
from jax.experimental.pallas import tpu_sc as plsc

SEMI_LAMBDA = 0.5
N_NODES = 100000
N_EDGES = 6400000
N_CLASSES = 16

NC = 2
NS = 16
NW = NC * NS
L = 16
NB = N_EDGES // 128
VPB = 128 // L
CBLK = 25
CHUNK_W = CBLK * 256
FULL = 62
TBLK = 13
TAIL_W = TBLK * 256

_sc_mesh = plsc.VectorSubcoreMesh(core_axis_name="c", subcore_axis_name="s")


@functools.partial(
    pl.kernel,
    mesh=_sc_mesh,
    compiler_params=pltpu.CompilerParams(needs_layout_passes=False),
    out_type=jax.ShapeDtypeStruct((NC, NS, L), jnp.float32),
    scratch_types=[
        pltpu.VMEM((N_NODES,), jnp.int32),
        pltpu.VMEM((CHUNK_W,), jnp.int32),
        pltpu.VMEM((CHUNK_W,), jnp.int32),
        pltpu.VMEM((L,), jnp.float32),
        pltpu.SemaphoreType.DMA,
        pltpu.SemaphoreType.DMA,
    ],
)
def _sc_edge(ptab_hbm, edges_hbm, out_hbm, table_v, ebuf0, ebuf1, accv, sem_t, sem_e):
    c = lax.axis_index("c")
    s = lax.axis_index("s")
    w = s * NC + c
    pltpu.async_copy(ptab_hbm, table_v, sem_t).wait()

    nblk = 1562 + jnp.where(w < 16, 1, 0)
    base_blk = 1562 * w + jnp.minimum(w, 16)
    base = base_blk * 256
    bufs = (ebuf0, ebuf1)

    def start(off_w, nwords, buf):
        pltpu.async_copy(
            edges_hbm.at[pl.ds(off_w, nwords)], bufs[buf].at[pl.ds(0, nwords)], sem_e
        )

    def wait(off_w, nwords, buf):
        pltpu.make_async_copy(
            edges_hbm.at[pl.ds(off_w, nwords)], bufs[buf].at[pl.ds(0, nwords)], sem_e
        ).wait()

    start(base, CHUNK_W, 0)

    def make_step(eb):
        def body(blk, acc):
            off = blk * 256
            for t in range(VPB):
                srcs = eb[pl.ds(off + t * L, L)]
                dsts = eb[pl.ds(off + 128 + t * L, L)]
                ws = plsc.load_gather(table_v, [srcs])
                wd = plsc.load_gather(table_v, [dsts])
                a0, a1 = plsc.unpack(
                    plsc.bitcast(ws, jnp.bfloat16), format=plsc.PackFormat.INTERLEAVED
                )
                b0, b1 = plsc.unpack(
                    plsc.bitcast(wd, jnp.bfloat16), format=plsc.PackFormat.INTERLEAVED
                )
                d0 = a0 - b0
                d1 = a1 - b1
                acc = acc + d0 * d0
                acc = acc + d1 * d1
            return acc

        return body

    def do_chunk(g, b, acc):
        @pl.when(g + 1 < FULL)
        def _():
            start(base + (g + 1) * CHUNK_W, CHUNK_W, 1 - b)

        @pl.when(g + 1 == FULL)
        def _():
            start((base_blk + nblk - TBLK) * 256, TAIL_W, 1 - b)

        wait(base + g * CHUNK_W, CHUNK_W, b)
        return lax.fori_loop(0, CBLK, make_step(bufs[b]), acc)

    def outer(g2, acc):
        acc = do_chunk(2 * g2, 0, acc)
        acc = do_chunk(2 * g2 + 1, 1, acc)
        return acc

    acc = lax.fori_loop(0, FULL // 2, outer, jnp.zeros((L,), jnp.float32))
    wait((base_blk + nblk - TBLK) * 256, TAIL_W, 0)
    j0 = 1563 - nblk
    acc = lax.fori_loop(j0, TBLK, make_step(ebuf0), acc)
    accv[...] = acc
    pltpu.sync_copy(accv, out_hbm.at[c, s])


def _tc_body(pn_ref, gt_ref, m_ref, pe_ref, out_ref):
    p = pn_ref[...]
    gt = gt_ref[...]
    m = m_ref[...].astype(jnp.float32)
    rows = lax.broadcasted_iota(jnp.int32, (N_CLASSES, N_NODES), 0)
    pv = jnp.sum(jnp.where(rows == gt[None, :], p, 0.0), axis=0)
    nll = -jnp.sum(jnp.log(pv) * m) / jnp.sum(m)
    t = jnp.float32(N_NODES) - jnp.sum(pe_ref[1, :])
    out_ref[0] = nll
    out_ref[1] = t


_tc_nll = pl.pallas_call(
    _tc_body,
    out_specs=pl.BlockSpec(memory_space=pltpu.SMEM),
    out_shape=jax.ShapeDtypeStruct((2,), jnp.float32),
)


def kernel(poss_node, poss_edge, weights, groundTruth, mask, edges):
    pe_t = poss_edge.T
    pn_t = poss_node.T
    gt = groundTruth.astype(jnp.int32)
    m = mask.astype(jnp.int32)
    edges_n = edges.reshape(N_EDGES // 128, 128, 2).transpose(0, 2, 1).reshape(-1)
    ptab = jax.lax.bitcast_convert_type(poss_edge.astype(jnp.bfloat16), jnp.int32)
    parts = _sc_edge(ptab, edges_n)
    tc = _tc_nll(pn_t, gt, m, pe_t)
    s_total = jnp.sum(parts)
    return tc[0] + jnp.float32(SEMI_LAMBDA) * s_total * tc[1]


# speedup vs baseline: 606.3498x; 1.0168x over previous
import functools

import jax
import jax.numpy as jnp
from jax import lax
from jax.experimental import pallas as pl
from jax.experimental.pallas import tpu as pltpu
from jax.experimental.pallas import tpu_sc as plsc

SEMI_LAMBDA = 0.5
N_NODES = 100000
N_EDGES = 6400000
N_CLASSES = 16

NC = 2
NS = 16
NW = NC * NS
L = 16
NB = N_EDGES // 128
VPB = 128 // L
CBLK = 25
CHUNK_W = CBLK * 256
FULL = 62
TBLK = 13
TAIL_W = TBLK * 256

_sc_mesh = plsc.VectorSubcoreMesh(core_axis_name="c", subcore_axis_name="s")


@functools.partial(
    pl.kernel,
    mesh=_sc_mesh,
    compiler_params=pltpu.CompilerParams(needs_layout_passes=False),
    out_type=jax.ShapeDtypeStruct((NC, NS, L), jnp.float32),
    scratch_types=[
        pltpu.VMEM((N_NODES,), jnp.int32),
        pltpu.VMEM((CHUNK_W,), jnp.int32),
        pltpu.VMEM((CHUNK_W,), jnp.int32),
        pltpu.VMEM((L,), jnp.float32),
        pltpu.SemaphoreType.DMA,
        pltpu.SemaphoreType.DMA,
    ],
)
def _sc_edge(ptab_hbm, edges_hbm, out_hbm, table_v, ebuf0, ebuf1, accv, sem_t, sem_e):
    c = lax.axis_index("c")
    s = lax.axis_index("s")
    w = s * NC + c
    table_cp = pltpu.async_copy(ptab_hbm, table_v, sem_t)

    nblk = 1562 + jnp.where(w < 16, 1, 0)
    base_blk = 1562 * w + jnp.minimum(w, 16)
    base = base_blk * 256
    bufs = (ebuf0, ebuf1)

    def start(off_w, nwords, buf):
        pltpu.async_copy(
            edges_hbm.at[pl.ds(off_w, nwords)], bufs[buf].at[pl.ds(0, nwords)], sem_e
        )

    def wait(off_w, nwords, buf):
        pltpu.make_async_copy(
            edges_hbm.at[pl.ds(off_w, nwords)], bufs[buf].at[pl.ds(0, nwords)], sem_e
        ).wait()

    start(base, CHUNK_W, 0)
    table_cp.wait()

    def make_step(eb):
        def body(blk, acc):
            off = blk * 256
            for t in range(VPB):
                srcs = eb[pl.ds(off + t * L, L)]
                dsts = eb[pl.ds(off + 128 + t * L, L)]
                ws = plsc.load_gather(table_v, [srcs])
                wd = plsc.load_gather(table_v, [dsts])
                a0, a1 = plsc.unpack(
                    plsc.bitcast(ws, jnp.bfloat16), format=plsc.PackFormat.INTERLEAVED
                )
                b0, b1 = plsc.unpack(
                    plsc.bitcast(wd, jnp.bfloat16), format=plsc.PackFormat.INTERLEAVED
                )
                d0 = a0 - b0
                d1 = a1 - b1
                acc = acc + d0 * d0
                acc = acc + d1 * d1
            return acc

        return body

    def do_chunk(g, b, acc):
        @pl.when(g + 1 < FULL)
        def _():
            start(base + (g + 1) * CHUNK_W, CHUNK_W, 1 - b)

        @pl.when(g + 1 == FULL)
        def _():
            start((base_blk + nblk - TBLK) * 256, TAIL_W, 1 - b)

        wait(base + g * CHUNK_W, CHUNK_W, b)
        return plsc.parallel_loop(0, CBLK, unroll=4, carry=acc)(make_step(bufs[b]))

    def outer(g2, acc):
        acc = do_chunk(2 * g2, 0, acc)
        acc = do_chunk(2 * g2 + 1, 1, acc)
        return acc

    acc = lax.fori_loop(0, FULL // 2, outer, jnp.zeros((L,), jnp.float32))
    wait((base_blk + nblk - TBLK) * 256, TAIL_W, 0)
    j0 = 1563 - nblk
    acc = plsc.parallel_loop(j0, TBLK, carry=acc)(make_step(ebuf0))
    accv[...] = acc
    pltpu.sync_copy(accv, out_hbm.at[c, s])


def _tc_body(pn_ref, gt_ref, m_ref, pe_ref, out_ref):
    p = pn_ref[...]
    gt = gt_ref[...]
    m = m_ref[...].astype(jnp.float32)
    rows = lax.broadcasted_iota(jnp.int32, (N_CLASSES, N_NODES), 0)
    pv = jnp.sum(jnp.where(rows == gt[None, :], p, 0.0), axis=0)
    nll = -jnp.sum(jnp.log(pv) * m) / jnp.sum(m)
    t = jnp.float32(N_NODES) - jnp.sum(pe_ref[1, :])
    out_ref[0] = nll
    out_ref[1] = t


_tc_nll = pl.pallas_call(
    _tc_body,
    out_specs=pl.BlockSpec(memory_space=pltpu.SMEM),
    out_shape=jax.ShapeDtypeStruct((2,), jnp.float32),
)


def kernel(poss_node, poss_edge, weights, groundTruth, mask, edges):
    pe_t = poss_edge.T
    pn_t = poss_node.T
    gt = groundTruth.astype(jnp.int32)
    m = mask.astype(jnp.int32)
    edges_n = edges.reshape(N_EDGES // 128, 128, 2).transpose(0, 2, 1).reshape(-1)
    ptab = jax.lax.bitcast_convert_type(poss_edge.astype(jnp.bfloat16), jnp.int32)
    parts = _sc_edge(ptab, edges_n)
    tc = _tc_nll(pn_t, gt, m, pe_t)
    s_total = jnp.sum(parts)
    return tc[0] + jnp.float32(SEMI_LAMBDA) * s_total * tc[1]


# speedup vs baseline: 611.6161x; 1.0087x over previous
import functools

import jax
import jax.numpy as jnp
from jax import lax
from jax.experimental import pallas as pl
from jax.experimental.pallas import tpu as pltpu
from jax.experimental.pallas import tpu_sc as plsc

SEMI_LAMBDA = 0.5
N_NODES = 100000
N_EDGES = 6400000
N_CLASSES = 16

NC = 2
NS = 16
NW = NC * NS
L = 16
NB = N_EDGES // 128
VPB = 128 // L
CBLK = 25
CHUNK_W = CBLK * 256
FULL = 62
TBLK = 13
TAIL_W = TBLK * 256

_sc_mesh = plsc.VectorSubcoreMesh(core_axis_name="c", subcore_axis_name="s")


@functools.partial(
    pl.kernel,
    mesh=_sc_mesh,
    compiler_params=pltpu.CompilerParams(needs_layout_passes=False),
    out_type=jax.ShapeDtypeStruct((NC, NS, L), jnp.float32),
    scratch_types=[
        pltpu.VMEM((N_NODES,), jnp.int32),
        pltpu.VMEM((CHUNK_W,), jnp.int32),
        pltpu.VMEM((CHUNK_W,), jnp.int32),
        pltpu.VMEM((L,), jnp.float32),
        pltpu.SemaphoreType.DMA,
        pltpu.SemaphoreType.DMA,
    ],
)
def _sc_edge(ptab_hbm, edges_hbm, out_hbm, table_v, ebuf0, ebuf1, accv, sem_t, sem_e):
    c = lax.axis_index("c")
    s = lax.axis_index("s")
    w = s * NC + c
    table_cp = pltpu.async_copy(ptab_hbm, table_v, sem_t)

    nblk = 1562 + jnp.where(w < 16, 1, 0)
    base_blk = 1562 * w + jnp.minimum(w, 16)
    base = base_blk * 256
    bufs = (ebuf0, ebuf1)

    def start(off_w, nwords, buf):
        pltpu.async_copy(
            edges_hbm.at[pl.ds(off_w, nwords)], bufs[buf].at[pl.ds(0, nwords)], sem_e
        )

    def wait(off_w, nwords, buf):
        pltpu.make_async_copy(
            edges_hbm.at[pl.ds(off_w, nwords)], bufs[buf].at[pl.ds(0, nwords)], sem_e
        ).wait()

    start(base, CHUNK_W, 0)
    table_cp.wait()

    def make_step(eb):
        def body(blk, acc):
            off = blk * 256
            for t in range(VPB):
                srcs = eb[pl.ds(off + t * L, L)]
                dsts = eb[pl.ds(off + 128 + t * L, L)]
                ws = plsc.load_gather(table_v, [srcs])
                wd = plsc.load_gather(table_v, [dsts])
                a0, a1 = plsc.unpack(
                    plsc.bitcast(ws, jnp.bfloat16), format=plsc.PackFormat.INTERLEAVED
                )
                b0, b1 = plsc.unpack(
                    plsc.bitcast(wd, jnp.bfloat16), format=plsc.PackFormat.INTERLEAVED
                )
                d0 = a0 - b0
                d1 = a1 - b1
                acc = acc + d0 * d0
                acc = acc + d1 * d1
            return acc

        return body

    def do_chunk(g, b, acc):
        @pl.when(g + 1 < FULL)
        def _():
            start(base + (g + 1) * CHUNK_W, CHUNK_W, 1 - b)

        @pl.when(g + 1 == FULL)
        def _():
            start((base_blk + nblk - TBLK) * 256, TAIL_W, 1 - b)

        wait(base + g * CHUNK_W, CHUNK_W, b)
        return plsc.parallel_loop(0, CBLK, unroll=4, carry=acc)(make_step(bufs[b]))

    def outer(g2, acc):
        acc = do_chunk(2 * g2, 0, acc)
        acc = do_chunk(2 * g2 + 1, 1, acc)
        return acc

    acc = lax.fori_loop(0, FULL // 2, outer, jnp.zeros((L,), jnp.float32))
    wait((base_blk + nblk - TBLK) * 256, TAIL_W, 0)
    j0 = 1563 - nblk
    acc = plsc.parallel_loop(j0, TBLK, carry=acc)(make_step(ebuf0))
    accv[...] = acc
    pltpu.sync_copy(accv, out_hbm.at[c, s])


def _tc_body(pn_ref, gt_ref, m_ref, pe_ref, out_ref):
    p = pn_ref[...]
    gt = gt_ref[...]
    m = m_ref[...].astype(jnp.float32)
    rows = lax.broadcasted_iota(jnp.int32, (N_CLASSES, N_NODES), 0)
    pv = jnp.sum(jnp.where(rows == gt[None, :], p, 0.0), axis=0)
    nll = -jnp.sum(jnp.log(pv) * m) / jnp.sum(m)
    t = jnp.float32(N_NODES) - jnp.sum(pe_ref[1, :])
    out_ref[0] = nll
    out_ref[1] = t


_tc_nll = pl.pallas_call(
    _tc_body,
    out_specs=pl.BlockSpec(memory_space=pltpu.SMEM),
    out_shape=jax.ShapeDtypeStruct((2,), jnp.float32),
)


def _combine_body(parts_ref, tc_ref, out_ref):
    s = jnp.sum(parts_ref[...])
    out_ref[0] = tc_ref[0] + jnp.float32(SEMI_LAMBDA) * s * tc_ref[1]


_tc_combine = pl.pallas_call(
    _combine_body,
    in_specs=[
        pl.BlockSpec(memory_space=pltpu.VMEM),
        pl.BlockSpec(memory_space=pltpu.SMEM),
    ],
    out_specs=pl.BlockSpec(memory_space=pltpu.SMEM),
    out_shape=jax.ShapeDtypeStruct((1,), jnp.float32),
)


def kernel(poss_node, poss_edge, weights, groundTruth, mask, edges):
    pe_t = poss_edge.T
    pn_t = poss_node.T
    gt = groundTruth.astype(jnp.int32)
    m = mask.astype(jnp.int32)
    edges_n = edges.reshape(N_EDGES // 128, 128, 2).transpose(0, 2, 1).reshape(-1)
    pe_bf = poss_edge.astype(jnp.bfloat16)
    h0 = jax.lax.bitcast_convert_type(pe_bf[:, 0], jnp.uint16).astype(jnp.int32)
    h1 = jax.lax.bitcast_convert_type(pe_bf[:, 1], jnp.uint16).astype(jnp.int32)
    ptab = h0 | (h1 << 16)
    parts = _sc_edge(ptab, edges_n)
    tc = _tc_nll(pn_t, gt, m, pe_t)
    return jnp.reshape(_tc_combine(parts, tc), ())


# speedup vs baseline: 613.1448x; 1.0025x over previous
import functools

import jax
import jax.numpy as jnp
from jax import lax
from jax.experimental import pallas as pl
from jax.experimental.pallas import tpu as pltpu
from jax.experimental.pallas import tpu_sc as plsc

SEMI_LAMBDA = 0.5
N_NODES = 100000
N_EDGES = 6400000
N_CLASSES = 16

NC = 2
NS = 16
NW = NC * NS
L = 16
NB = N_EDGES // 128
VPB = 128 // L
CBLK = 25
CHUNK_W = CBLK * 256
FULL = 62
TBLK = 13
TAIL_W = TBLK * 256

_sc_mesh = plsc.VectorSubcoreMesh(core_axis_name="c", subcore_axis_name="s")


@functools.partial(
    pl.kernel,
    mesh=_sc_mesh,
    compiler_params=pltpu.CompilerParams(needs_layout_passes=False),
    out_type=jax.ShapeDtypeStruct((NC, NS, L), jnp.float32),
    scratch_types=[
        pltpu.VMEM((N_NODES,), jnp.int32),
        pltpu.VMEM((CHUNK_W,), jnp.int32),
        pltpu.VMEM((CHUNK_W,), jnp.int32),
        pltpu.VMEM((L,), jnp.float32),
        pltpu.SemaphoreType.DMA,
        pltpu.SemaphoreType.DMA,
    ],
)
def _sc_edge(ptab_hbm, edges_hbm, out_hbm, table_v, ebuf0, ebuf1, accv, sem_t, sem_e):
    c = lax.axis_index("c")
    s = lax.axis_index("s")
    w = s * NC + c
    table_cp = pltpu.async_copy(ptab_hbm, table_v, sem_t)

    nblk = 1562 + jnp.where(w < 16, 1, 0)
    base_blk = 1562 * w + jnp.minimum(w, 16)
    base = base_blk * 256
    bufs = (ebuf0, ebuf1)

    def start(off_w, nwords, buf):
        pltpu.async_copy(
            edges_hbm.at[pl.ds(off_w, nwords)], bufs[buf].at[pl.ds(0, nwords)], sem_e
        )

    def wait(off_w, nwords, buf):
        pltpu.make_async_copy(
            edges_hbm.at[pl.ds(off_w, nwords)], bufs[buf].at[pl.ds(0, nwords)], sem_e
        ).wait()

    start(base, CHUNK_W, 0)
    table_cp.wait()

    def make_step(eb):
        def body(blk, acc):
            off = blk * 256
            for t in range(VPB):
                srcs = eb[pl.ds(off + t * L, L)]
                dsts = eb[pl.ds(off + 128 + t * L, L)]
                ws = plsc.load_gather(table_v, [srcs])
                wd = plsc.load_gather(table_v, [dsts])
                db = plsc.bitcast(ws, jnp.bfloat16) - plsc.bitcast(wd, jnp.bfloat16)
                d0, d1 = plsc.unpack(db, format=plsc.PackFormat.INTERLEAVED)
                acc = acc + d0 * d0
                acc = acc + d1 * d1
            return acc

        return body

    def do_chunk(g, b, acc):
        @pl.when(g + 1 < FULL)
        def _():
            start(base + (g + 1) * CHUNK_W, CHUNK_W, 1 - b)

        @pl.when(g + 1 == FULL)
        def _():
            start((base_blk + nblk - TBLK) * 256, TAIL_W, 1 - b)

        wait(base + g * CHUNK_W, CHUNK_W, b)
        return plsc.parallel_loop(0, CBLK, unroll=4, carry=acc)(make_step(bufs[b]))

    def outer(g2, acc):
        acc = do_chunk(2 * g2, 0, acc)
        acc = do_chunk(2 * g2 + 1, 1, acc)
        return acc

    acc = lax.fori_loop(0, FULL // 2, outer, jnp.zeros((L,), jnp.float32))
    wait((base_blk + nblk - TBLK) * 256, TAIL_W, 0)
    j0 = 1563 - nblk
    acc = plsc.parallel_loop(j0, TBLK, carry=acc)(make_step(ebuf0))
    accv[...] = acc
    pltpu.sync_copy(accv, out_hbm.at[c, s])


def _tc_body(pn_ref, gt_ref, m_ref, pe_ref, out_ref):
    p = pn_ref[...]
    gt = gt_ref[...]
    m = m_ref[...].astype(jnp.float32)
    rows = lax.broadcasted_iota(jnp.int32, (N_CLASSES, N_NODES), 0)
    pv = jnp.sum(jnp.where(rows == gt[None, :], p, 0.0), axis=0)
    nll = -jnp.sum(jnp.log(pv) * m) / jnp.sum(m)
    t = jnp.float32(N_NODES) - jnp.sum(pe_ref[1, :])
    out_ref[0] = nll
    out_ref[1] = t


_tc_nll = pl.pallas_call(
    _tc_body,
    out_specs=pl.BlockSpec(memory_space=pltpu.SMEM),
    out_shape=jax.ShapeDtypeStruct((2,), jnp.float32),
)


def _combine_body(parts_ref, tc_ref, out_ref):
    s = jnp.sum(parts_ref[...])
    out_ref[0] = tc_ref[0] + jnp.float32(SEMI_LAMBDA) * s * tc_ref[1]


_tc_combine = pl.pallas_call(
    _combine_body,
    in_specs=[
        pl.BlockSpec(memory_space=pltpu.VMEM),
        pl.BlockSpec(memory_space=pltpu.SMEM),
    ],
    out_specs=pl.BlockSpec(memory_space=pltpu.SMEM),
    out_shape=jax.ShapeDtypeStruct((1,), jnp.float32),
)


def kernel(poss_node, poss_edge, weights, groundTruth, mask, edges):
    pe_t = poss_edge.T
    pn_t = poss_node.T
    gt = groundTruth.astype(jnp.int32)
    m = mask.astype(jnp.int32)
    edges_n = edges.reshape(N_EDGES // 128, 128, 2).transpose(0, 2, 1).reshape(-1)
    pe_bf = poss_edge.astype(jnp.bfloat16)
    h0 = jax.lax.bitcast_convert_type(pe_bf[:, 0], jnp.uint16).astype(jnp.int32)
    h1 = jax.lax.bitcast_convert_type(pe_bf[:, 1], jnp.uint16).astype(jnp.int32)
    ptab = h0 | (h1 << 16)
    parts = _sc_edge(ptab, edges_n)
    tc = _tc_nll(pn_t, gt, m, pe_t)
    return jnp.reshape(_tc_combine(parts, tc), ())
